# baseline (speedup 1.0000x reference)
import jax
import jax.numpy as jnp
from jax.experimental import pallas as pl
from jax.experimental.pallas import tpu as pltpu
from jax.experimental.pallas import tpu_sc as plsc

_BR = 8
_BC = 512
_LANES = 16


def kernel(x, emb_weight):
    batch, num_patches, dim = x.shape
    rows = batch * num_patches
    nbe = num_patches // _BR
    x2 = x.reshape(rows, dim)

    mesh = plsc.VectorSubcoreMesh(core_axis_name="c", subcore_axis_name="s")

    @pl.kernel(out_type=jax.ShapeDtypeStruct((rows, dim), x.dtype), mesh=mesh)
    def sc_kernel(x_hbm, emb_hbm, o_hbm):
        def body(x_vmem, emb_vmem, o_vmem):
            @pl.loop(0, _BR)
            def _(r):
                @pl.loop(0, _BC, step=_LANES)
                def _(c):
                    o_vmem.at[pl.ds(r, 1), pl.ds(c, _LANES)][...] = (
                        x_vmem.at[pl.ds(r, 1), pl.ds(c, _LANES)][...]
                        + emb_vmem.at[pl.ds(r, 1), pl.ds(c, _LANES)][...]
                    )

        pltpu.emit_pipeline(
            body,
            grid=(rows // _BR, dim // _BC),
            in_specs=[
                pl.BlockSpec((_BR, _BC), lambda i, j: (i, j)),
                pl.BlockSpec((_BR, _BC), lambda i, j: (i % nbe, j)),
            ],
            out_specs=[pl.BlockSpec((_BR, _BC), lambda i, j: (i, j))],
            core_axis_name=("c", "s"),
            dimension_semantics=(pltpu.PARALLEL, pltpu.PARALLEL),
        )(x_hbm, emb_hbm, o_hbm)

    return sc_kernel(x2, emb_weight).reshape(x.shape)


# speedup vs baseline: 1.0077x; 1.0077x over previous
import jax
import jax.numpy as jnp
from jax.experimental import pallas as pl
from jax.experimental.pallas import tpu as pltpu
from jax.experimental.pallas import tpu_sc as plsc

_BR = 8
_BC = 512
_LANES = 16
_UNROLL = 4


def kernel(x, emb_weight):
    batch, num_patches, dim = x.shape

    mesh = plsc.VectorSubcoreMesh(core_axis_name="c", subcore_axis_name="s")

    @pl.kernel(out_type=jax.ShapeDtypeStruct(x.shape, x.dtype), mesh=mesh)
    def sc_kernel(x_hbm, emb_hbm, o_hbm):
        def body(x_vmem, emb_vmem, o_vmem):
            @pl.loop(0, _BR)
            def _(r):
                @pl.loop(0, _BC, step=_LANES * _UNROLL)
                def _(c):
                    for u in range(_UNROLL):
                        cs = pl.ds(c + u * _LANES, _LANES)
                        e = emb_vmem.at[pl.ds(r, 1), cs][...]
                        for b in range(batch):
                            o_vmem.at[b, pl.ds(r, 1), cs][...] = (
                                x_vmem.at[b, pl.ds(r, 1), cs][...] + e
                            )

        pltpu.emit_pipeline(
            body,
            grid=(num_patches // _BR, dim // _BC),
            in_specs=[
                pl.BlockSpec((batch, _BR, _BC), lambda i, j: (0, i, j)),
                pl.BlockSpec((_BR, _BC), lambda i, j: (i, j)),
            ],
            out_specs=[pl.BlockSpec((batch, _BR, _BC), lambda i, j: (0, i, j))],
            core_axis_name=("c", "s"),
            dimension_semantics=(pltpu.PARALLEL, pltpu.PARALLEL),
        )(x_hbm, emb_hbm, o_hbm)

    return sc_kernel(x, emb_weight)
